# baseline (speedup 1.0000x reference)
import functools

import jax
import jax.numpy as jnp
from jax import lax
from jax.experimental import pallas as pl
from jax.experimental.pallas import tpu as pltpu
from jax.experimental.pallas import tpu_sc as plsc

N = 10000
D = 128
E = 320000

NC = 2
NS = 16
NW = NC * NS

CHUNK = 64
CHUNKS = 160
EPT = CHUNKS * CHUNK
EPAD = NW * EPT
HALF = CHUNKS // 4
STEPS = HALF // 5

NPAD = 10112
ZROWS = NPAD // NS
DUMMY = N


def _mp_body(h_hbm, ed_hbm, out_hbm, ed_v,
             buf0, buf1, buf2, buf3, buf4, acc,
             sg0, sg1, sg2, sg3, sg4, ss0, ss1, ss2, ss3, ss4):
    c = lax.axis_index("c")
    s = lax.axis_index("s")
    w = c * NS + s
    bufs = (buf0, buf1, buf2, buf3, buf4)
    sgs = (sg0, sg1, sg2, sg3, sg4)
    sss = (ss0, ss1, ss2, ss3, ss4)

    def _z(i, _):
        r = i // 8
        col = (i % 8) * 16
        buf0[r, pl.ds(col, 16)] = jnp.zeros((16,), jnp.float32)
        return 0

    lax.fori_loop(0, 64 * 8, _z, 0)
    for k in range(ZROWS // CHUNK):
        pltpu.sync_copy(buf0, acc.at[pl.ds(s * ZROWS + k * CHUNK, CHUNK)])
    pltpu.sync_copy(buf0.at[pl.ds(0, ZROWS % CHUNK)],
                    acc.at[pl.ds(s * ZROWS + (ZROWS // CHUNK) * CHUNK,
                                 ZROWS % CHUNK)])
    plsc.subcore_barrier()

    def _gather(m, b, sem):
        pltpu.async_copy(h_hbm.at[ed_v.at[m, pl.ds(0, CHUNK)]], bufs[b], sem)

    def _wait(b, sem):
        pltpu.make_async_copy(h_hbm.at[pl.ds(0, CHUNK)], bufs[b], sem).wait()

    def _scatter(m, b, sem):
        pltpu.async_copy(bufs[b], acc.at[ed_v.at[m, pl.ds(CHUNK, CHUNK)]],
                         sem, add=True)

    for h in range(CHUNKS // HALF):
        pltpu.sync_copy(ed_hbm.at[w, pl.ds(h * HALF, HALF)], ed_v)
        _gather(0, 0, sg0)
        _gather(1, 1, sg1)

        def _step(k, _):
            for p in range(5):
                m = 5 * k + p
                gb = (p + 2) % 5

                _wait(p, sgs[p])
                _scatter(m, p, sss[p])

                if p < 3:
                    @pl.when(k > 0)
                    def _():
                        _wait(gb, sss[gb])
                    _gather(m + 2, gb, sgs[gb])
                else:
                    _wait(gb, sss[gb])

                    @pl.when(k + 1 < STEPS)
                    def _():
                        _gather(m + 2, gb, sgs[gb])
            return 0

        lax.fori_loop(0, STEPS, _step, 0)
        _wait(2, ss2)
        _wait(3, ss3)
        _wait(4, ss4)
    plsc.subcore_barrier()

    pltpu.sync_copy(acc.at[pl.ds(s * ZROWS, ZROWS)],
                    out_hbm.at[c, pl.ds(s * ZROWS, ZROWS)])


_mp_kernel = functools.partial(
    pl.kernel,
    mesh=plsc.VectorSubcoreMesh(core_axis_name="c", subcore_axis_name="s"),
    out_type=jax.ShapeDtypeStruct((NC, NPAD, D), jnp.float32),
    scratch_types=[
        pltpu.VMEM((HALF, 2 * CHUNK), jnp.int32),
        pltpu.VMEM((CHUNK, D), jnp.float32),
        pltpu.VMEM((CHUNK, D), jnp.float32),
        pltpu.VMEM((CHUNK, D), jnp.float32),
        pltpu.VMEM((CHUNK, D), jnp.float32),
        pltpu.VMEM((CHUNK, D), jnp.float32),
        pltpu.VMEM_SHARED((NPAD, D), jnp.float32),
        pltpu.SemaphoreType.DMA,
        pltpu.SemaphoreType.DMA,
        pltpu.SemaphoreType.DMA,
        pltpu.SemaphoreType.DMA,
        pltpu.SemaphoreType.DMA,
        pltpu.SemaphoreType.DMA,
        pltpu.SemaphoreType.DMA,
        pltpu.SemaphoreType.DMA,
        pltpu.SemaphoreType.DMA,
        pltpu.SemaphoreType.DMA,
    ],
)(_mp_body)


def _message_pass(h, ed3):
    return _mp_kernel(h, ed3)


ROWS_BLK = 2000


def _mm_body(x_ref, w_ref, o_ref):
    o_ref[...] = jnp.dot(x_ref[...], w_ref[...],
                         preferred_element_type=jnp.float32)


def _mm(x, w):
    return pl.pallas_call(
        _mm_body,
        grid=(N // ROWS_BLK,),
        in_specs=[
            pl.BlockSpec((ROWS_BLK, D), lambda i: (i, 0)),
            pl.BlockSpec((D, D), lambda i: (0, 0)),
        ],
        out_specs=pl.BlockSpec((ROWS_BLK, D), lambda i: (i, 0)),
        out_shape=jax.ShapeDtypeStruct((N, D), jnp.float32),
    )(x, w)


def _fuse_body(a_ref, b_ref, w_ref, o_ref):
    t = jnp.maximum(a_ref[0] + a_ref[1] + b_ref[...], 0.0)
    o_ref[...] = jnp.dot(t, w_ref[...], preferred_element_type=jnp.float32)


def _fuse_mm(acc, b, w):
    return pl.pallas_call(
        _fuse_body,
        grid=(N // ROWS_BLK,),
        in_specs=[
            pl.BlockSpec((NC, ROWS_BLK, D), lambda i: (0, i, 0)),
            pl.BlockSpec((1, D), lambda i: (0, 0)),
            pl.BlockSpec((D, D), lambda i: (0, 0)),
        ],
        out_specs=pl.BlockSpec((ROWS_BLK, D), lambda i: (i, 0)),
        out_shape=jax.ShapeDtypeStruct((N, D), jnp.float32),
    )(acc, b.reshape(1, D), w)


def _final_body(a_ref, b_ref, o_ref):
    o_ref[...] = jnp.maximum(a_ref[0] + a_ref[1] + b_ref[...], 0.0)


def _final(acc, b):
    return pl.pallas_call(
        _final_body,
        grid=(N // ROWS_BLK,),
        in_specs=[
            pl.BlockSpec((NC, ROWS_BLK, D), lambda i: (0, i, 0)),
            pl.BlockSpec((1, D), lambda i: (0, 0)),
        ],
        out_specs=pl.BlockSpec((ROWS_BLK, D), lambda i: (i, 0)),
        out_shape=jax.ShapeDtypeStruct((N, D), jnp.float32),
    )(acc, b.reshape(1, D))


def kernel(x, edge_index, W1, b1, W2, b2, W3, b3):
    src = edge_index[0].astype(jnp.int32)
    dst = edge_index[1].astype(jnp.int32)
    pad_i = jnp.arange(EPAD - E, dtype=jnp.int32) % CHUNK
    src3 = jnp.concatenate([src, pad_i]).reshape(NW, CHUNKS, CHUNK)
    dst3 = jnp.concatenate([dst, DUMMY + pad_i]).reshape(NW, CHUNKS, CHUNK)
    ed3 = jnp.concatenate([src3, dst3], axis=2)

    t = _mm(x, W1)
    acc = _message_pass(t, ed3)
    t = _fuse_mm(acc, b1, W2)
    acc = _message_pass(t, ed3)
    t = _fuse_mm(acc, b2, W3)
    acc = _message_pass(t, ed3)
    return _final(acc, b3)


# speedup vs baseline: 1.1445x; 1.1445x over previous
import functools

import jax
import jax.numpy as jnp
from jax import lax
from jax.experimental import pallas as pl
from jax.experimental.pallas import tpu as pltpu
from jax.experimental.pallas import tpu_sc as plsc

N = 10000
D = 128
E = 320000

NC = 2
NS = 16
NW = NC * NS

CHUNK = 64
CHUNKS = 160
EPT = CHUNKS * CHUNK
EPAD = NW * EPT
HALF = CHUNKS // 4
STEPS = HALF // 5

NPAD = 10112
ZROWS = NPAD // NS
DUMMY = N


def _mp_body(h_hbm, ed_hbm, out_hbm, ed_v,
             buf0, buf1, buf2, buf3, buf4, acc,
             sg0, sg1, sg2, sg3, sg4, ss0, ss1, ss2, ss3, ss4):
    c = lax.axis_index("c")
    s = lax.axis_index("s")
    w = c * NS + s
    bufs = (buf0, buf1, buf2, buf3, buf4)
    sgs = (sg0, sg1, sg2, sg3, sg4)
    sss = (ss0, ss1, ss2, ss3, ss4)

    def _z(i, _):
        r = i // 8
        col = (i % 8) * 16
        buf0[r, pl.ds(col, 16)] = jnp.zeros((16,), jnp.float32)
        return 0

    lax.fori_loop(0, 64 * 8, _z, 0)
    for k in range(ZROWS // CHUNK):
        pltpu.sync_copy(buf0, acc.at[pl.ds(s * ZROWS + k * CHUNK, CHUNK)])
    pltpu.sync_copy(buf0.at[pl.ds(0, ZROWS % CHUNK)],
                    acc.at[pl.ds(s * ZROWS + (ZROWS // CHUNK) * CHUNK,
                                 ZROWS % CHUNK)])
    plsc.subcore_barrier()

    def _gather(m, b, sem):
        pltpu.async_copy(h_hbm.at[ed_v.at[m, pl.ds(0, CHUNK)]], bufs[b], sem)

    def _wait(b, sem):
        pltpu.make_async_copy(h_hbm.at[pl.ds(0, CHUNK)], bufs[b], sem).wait()

    def _scatter(m, b, sem):
        pltpu.async_copy(bufs[b], acc.at[ed_v.at[m, pl.ds(CHUNK, CHUNK)]],
                         sem, add=True)

    for h in range(CHUNKS // HALF):
        pltpu.sync_copy(ed_hbm.at[w, pl.ds(h * HALF, HALF)], ed_v)
        _gather(0, 0, sg0)
        _gather(1, 1, sg1)

        def _step(k, _):
            for p in range(5):
                m = 5 * k + p
                gb = (p + 2) % 5

                if p < 3:
                    @pl.when(k > 0)
                    def _():
                        _wait(gb, sss[gb])
                    _gather(m + 2, gb, sgs[gb])
                else:
                    _wait(gb, sss[gb])

                    @pl.when(k + 1 < STEPS)
                    def _():
                        _gather(m + 2, gb, sgs[gb])

                _wait(p, sgs[p])
                _scatter(m, p, sss[p])
            return 0

        lax.fori_loop(0, STEPS, _step, 0)
        _wait(2, ss2)
        _wait(3, ss3)
        _wait(4, ss4)
    plsc.subcore_barrier()

    pltpu.sync_copy(acc.at[pl.ds(s * ZROWS, ZROWS)],
                    out_hbm.at[c, pl.ds(s * ZROWS, ZROWS)])


_mp_kernel = functools.partial(
    pl.kernel,
    mesh=plsc.VectorSubcoreMesh(core_axis_name="c", subcore_axis_name="s"),
    out_type=jax.ShapeDtypeStruct((NC, NPAD, D), jnp.float32),
    scratch_types=[
        pltpu.VMEM((HALF, 2 * CHUNK), jnp.int32),
        pltpu.VMEM((CHUNK, D), jnp.float32),
        pltpu.VMEM((CHUNK, D), jnp.float32),
        pltpu.VMEM((CHUNK, D), jnp.float32),
        pltpu.VMEM((CHUNK, D), jnp.float32),
        pltpu.VMEM((CHUNK, D), jnp.float32),
        pltpu.VMEM_SHARED((NPAD, D), jnp.float32),
        pltpu.SemaphoreType.DMA,
        pltpu.SemaphoreType.DMA,
        pltpu.SemaphoreType.DMA,
        pltpu.SemaphoreType.DMA,
        pltpu.SemaphoreType.DMA,
        pltpu.SemaphoreType.DMA,
        pltpu.SemaphoreType.DMA,
        pltpu.SemaphoreType.DMA,
        pltpu.SemaphoreType.DMA,
        pltpu.SemaphoreType.DMA,
    ],
)(_mp_body)


def _message_pass(h, ed3):
    return _mp_kernel(h, ed3)


ROWS_BLK = 2000


def _mm_body(x_ref, w_ref, o_ref):
    o_ref[...] = jnp.dot(x_ref[...], w_ref[...],
                         preferred_element_type=jnp.float32)


def _mm(x, w):
    return pl.pallas_call(
        _mm_body,
        grid=(N // ROWS_BLK,),
        in_specs=[
            pl.BlockSpec((ROWS_BLK, D), lambda i: (i, 0)),
            pl.BlockSpec((D, D), lambda i: (0, 0)),
        ],
        out_specs=pl.BlockSpec((ROWS_BLK, D), lambda i: (i, 0)),
        out_shape=jax.ShapeDtypeStruct((N, D), jnp.float32),
    )(x, w)


def _fuse_body(a_ref, b_ref, w_ref, o_ref):
    t = jnp.maximum(a_ref[0] + a_ref[1] + b_ref[...], 0.0)
    o_ref[...] = jnp.dot(t, w_ref[...], preferred_element_type=jnp.float32)


def _fuse_mm(acc, b, w):
    return pl.pallas_call(
        _fuse_body,
        grid=(N // ROWS_BLK,),
        in_specs=[
            pl.BlockSpec((NC, ROWS_BLK, D), lambda i: (0, i, 0)),
            pl.BlockSpec((1, D), lambda i: (0, 0)),
            pl.BlockSpec((D, D), lambda i: (0, 0)),
        ],
        out_specs=pl.BlockSpec((ROWS_BLK, D), lambda i: (i, 0)),
        out_shape=jax.ShapeDtypeStruct((N, D), jnp.float32),
    )(acc, b.reshape(1, D), w)


def _final_body(a_ref, b_ref, o_ref):
    o_ref[...] = jnp.maximum(a_ref[0] + a_ref[1] + b_ref[...], 0.0)


def _final(acc, b):
    return pl.pallas_call(
        _final_body,
        grid=(N // ROWS_BLK,),
        in_specs=[
            pl.BlockSpec((NC, ROWS_BLK, D), lambda i: (0, i, 0)),
            pl.BlockSpec((1, D), lambda i: (0, 0)),
        ],
        out_specs=pl.BlockSpec((ROWS_BLK, D), lambda i: (i, 0)),
        out_shape=jax.ShapeDtypeStruct((N, D), jnp.float32),
    )(acc, b.reshape(1, D))


def kernel(x, edge_index, W1, b1, W2, b2, W3, b3):
    src = edge_index[0].astype(jnp.int32)
    dst = edge_index[1].astype(jnp.int32)
    pad_i = jnp.arange(EPAD - E, dtype=jnp.int32) % CHUNK
    src3 = jnp.concatenate([src, pad_i]).reshape(NW, CHUNKS, CHUNK)
    dst3 = jnp.concatenate([dst, DUMMY + pad_i]).reshape(NW, CHUNKS, CHUNK)
    ed3 = jnp.concatenate([src3, dst3], axis=2)

    t = _mm(x, W1)
    acc = _message_pass(t, ed3)
    t = _fuse_mm(acc, b1, W2)
    acc = _message_pass(t, ed3)
    t = _fuse_mm(acc, b2, W3)
    acc = _message_pass(t, ed3)
    return _final(acc, b3)
